# baseline (speedup 1.0000x reference)
import jax
import jax.numpy as jnp
import numpy as np
from jax.experimental import pallas as pl
from jax.experimental.pallas import tpu as pltpu
from jax.experimental.pallas import tpu_sc as plsc

S = 2048
H = 512
NH = 8
DH = 64
E = 8
FF = 4 * H
L = 6
V = 32000

RB = 256
TB = 256
NBLK = S // TB + E
PADDED = NBLK * TB
QB = 512
VB = 1280

F32 = jnp.float32

_VMESH = plsc.VectorSubcoreMesh(core_axis_name="core",
                                subcore_axis_name="subcore")

_SC_CORES = 2
_SC_SUBCORES = 16
_WIN = 128


def _sc_gather(data, idx):
    n = idx.shape[0]
    width = data.shape[1]
    nwin = n // _WIN
    nsub = _SC_CORES * _SC_SUBCORES
    idx2 = idx.reshape(1, n)

    @pl.kernel(out_type=jax.ShapeDtypeStruct((n, width), data.dtype),
               mesh=_VMESH,
               scratch_types=[pltpu.VMEM((1, _WIN), jnp.int32),
                              pltpu.VMEM((_WIN, width), data.dtype),
                              pltpu.SemaphoreType.DMA])
    def k(x_hbm, i_hbm, o_hbm, ibuf, vbuf, sem):
        core = jax.lax.axis_index("core")
        sub = jax.lax.axis_index("subcore")
        wid = core * _SC_SUBCORES + sub

        @pl.loop(0, (nwin + nsub - 1) // nsub)
        def _(r):
            w = wid + r * nsub

            @pl.when(w < nwin)
            def _():
                pltpu.async_copy(i_hbm.at[:, pl.ds(w * _WIN, _WIN)],
                                 ibuf, sem).wait()
                pltpu.async_copy(x_hbm.at[ibuf.at[0]], vbuf, sem).wait()
                pltpu.async_copy(vbuf, o_hbm.at[pl.ds(w * _WIN, _WIN), :],
                                 sem).wait()

    return k(data, idx2)


def _sc_scatter(values, idx, out_rows):
    n, width = values.shape
    nwin = n // _WIN
    nsub = _SC_CORES * _SC_SUBCORES
    idx2 = idx.reshape(1, n)

    @pl.kernel(out_type=jax.ShapeDtypeStruct((out_rows, width),
                                             values.dtype),
               mesh=_VMESH,
               scratch_types=[pltpu.VMEM((1, _WIN), jnp.int32),
                              pltpu.VMEM((_WIN, width), values.dtype),
                              pltpu.SemaphoreType.DMA])
    def k(x_hbm, i_hbm, o_hbm, ibuf, vbuf, sem):
        core = jax.lax.axis_index("core")
        sub = jax.lax.axis_index("subcore")
        wid = core * _SC_SUBCORES + sub

        @pl.loop(0, (nwin + nsub - 1) // nsub)
        def _(r):
            w = wid + r * nsub

            @pl.when(w < nwin)
            def _():
                pltpu.async_copy(i_hbm.at[:, pl.ds(w * _WIN, _WIN)],
                                 ibuf, sem).wait()
                pltpu.async_copy(x_hbm.at[pl.ds(w * _WIN, _WIN), :],
                                 vbuf, sem).wait()
                pltpu.async_copy(vbuf, o_hbm.at[ibuf.at[0]], sem).wait()

    return k(values, idx2)


def _ln(x, g, b, eps=1e-5):
    m = jnp.mean(x, axis=-1, keepdims=True)
    v = jnp.mean((x - m) ** 2, axis=-1, keepdims=True)
    return (x - m) / jnp.sqrt(v + eps) * g + b


def _dot(a, b):
    return jax.lax.dot_general(a, b, (((1,), (1,)), ((), ())),
                               preferred_element_type=F32)


def _mm_body(x_ref, w_ref, b_ref, o_ref):
    o_ref[...] = _dot(x_ref[...], w_ref[...]) + b_ref[...]


def _mm(x, w, b):
    n = w.shape[0]
    return pl.pallas_call(
        _mm_body,
        grid=(S // RB,),
        in_specs=[
            pl.BlockSpec((RB, H), lambda i: (i, 0)),
            pl.BlockSpec((n, H), lambda i: (0, 0)),
            pl.BlockSpec((1, n), lambda i: (0, 0)),
        ],
        out_specs=pl.BlockSpec((RB, n), lambda i: (i, 0)),
        out_shape=jax.ShapeDtypeStruct((S, n), F32),
    )(x, w, b.reshape(1, n))


def _mm_nb_body(x_ref, w_ref, o_ref):
    o_ref[...] = _dot(x_ref[...], w_ref[...])


def _mm_nobias(x, w):
    n = w.shape[0]
    return pl.pallas_call(
        _mm_nb_body,
        grid=(S // RB,),
        in_specs=[
            pl.BlockSpec((RB, H), lambda i: (i, 0)),
            pl.BlockSpec((n, H), lambda i: (0, 0)),
        ],
        out_specs=pl.BlockSpec((RB, n), lambda i: (i, 0)),
        out_shape=jax.ShapeDtypeStruct((S, n), F32),
    )(x, w)


def _k2_body(q_ref, k_ref, o_ref):
    o_ref[0] = jax.lax.dot_general(
        q_ref[0], k_ref[0], (((1,), (1,)), ((), ())),
        preferred_element_type=F32) / np.float32(8.0)


def _attn_scores(qh, kh):
    return pl.pallas_call(
        _k2_body,
        grid=(NH, S // QB),
        in_specs=[
            pl.BlockSpec((1, QB, DH), lambda h, i: (h, i, 0)),
            pl.BlockSpec((1, S, DH), lambda h, i: (h, 0, 0)),
        ],
        out_specs=pl.BlockSpec((1, QB, S), lambda h, i: (h, i, 0)),
        out_shape=jax.ShapeDtypeStruct((NH, S, S), F32),
    )(qh, kh)


def _topk_oh(s):
    mx = jnp.max(s, axis=1, keepdims=True)
    iota_e = jax.lax.broadcasted_iota(jnp.int32, s.shape, 1)
    top1 = jnp.min(jnp.where(s == mx, iota_e, E), axis=1)
    oh = (iota_e == top1[:, None]).astype(F32)
    return mx, top1, oh


def _k4a_body(sc_ref, gate_out, rank_out, top1_out, cnt_out, ssum_out,
              acc_ref, sacc_ref):
    c = pl.program_id(0)

    @pl.when(c == 0)
    def _():
        acc_ref[...] = jnp.zeros_like(acc_ref)
        sacc_ref[...] = jnp.zeros_like(sacc_ref)

    s = sc_ref[...]
    mx, top1, oh = _topk_oh(s)
    gate_out[...] = mx / (mx + 1e-8)
    top1_out[...] = top1[:, None]
    ii = jax.lax.broadcasted_iota(jnp.int32, (RB, RB), 0)
    jj = jax.lax.broadcasted_iota(jnp.int32, (RB, RB), 1)
    stril = (jj < ii).astype(F32)
    ex = jax.lax.dot_general(stril, oh, (((1,), (0,)), ((), ())),
                             preferred_element_type=F32) + acc_ref[...]
    rank_out[...] = jnp.sum(ex * oh, axis=1)[:, None]
    acc_ref[...] += jnp.sum(oh, axis=0, keepdims=True)
    sacc_ref[...] += jnp.sum(s, axis=0, keepdims=True)
    cnt_out[...] = acc_ref[...]
    ssum_out[...] = sacc_ref[...]


def _k4b_body(cnt_ref, ssum_ref, top1_ref, rank_ref,
              gdest_out, blk_out, lb_out):
    counts = cnt_ref[...]
    pc = jnp.ceil(counts / TB) * TB
    ie0 = jax.lax.broadcasted_iota(jnp.int32, (E, E), 0)
    ie1 = jax.lax.broadcasted_iota(jnp.int32, (E, E), 1)
    tril_incl = (ie1 <= ie0).astype(F32)
    poff_incl = jnp.sum(tril_incl * pc, axis=1)
    poff = poff_incl - pc[0]
    top1 = top1_ref[...][:, 0]
    iota_e = jax.lax.broadcasted_iota(jnp.int32, (S, E), 1)
    oh = (iota_e == top1[:, None]).astype(F32)
    dest = jnp.sum(oh * poff[None, :], axis=1) + rank_ref[...][:, 0]
    gdest_out[...] = dest.astype(jnp.int32)[:, None]
    lb = E * jnp.sum((ssum_ref[...] / S) * (counts / S))
    lb_out[...] = lb.reshape(1, 1)
    bstart = (jax.lax.broadcasted_iota(jnp.int32, (NBLK, E), 0) * TB
              ).astype(F32)
    blk_e = jnp.sum((bstart >= poff_incl[None, :]).astype(jnp.int32), axis=1)
    blk_out[...] = jnp.minimum(blk_e, E - 1)[:, None]


def _route(scores):
    gate, rank, top1, cnt, ssum = pl.pallas_call(
        _k4a_body,
        grid=(S // RB,),
        in_specs=[pl.BlockSpec((RB, E), lambda c: (c, 0))],
        out_specs=[
            pl.BlockSpec((RB, 1), lambda c: (c, 0)),
            pl.BlockSpec((RB, 1), lambda c: (c, 0)),
            pl.BlockSpec((RB, 1), lambda c: (c, 0)),
            pl.BlockSpec((1, E), lambda c: (0, 0)),
            pl.BlockSpec((1, E), lambda c: (0, 0)),
        ],
        out_shape=[
            jax.ShapeDtypeStruct((S, 1), F32),
            jax.ShapeDtypeStruct((S, 1), F32),
            jax.ShapeDtypeStruct((S, 1), jnp.int32),
            jax.ShapeDtypeStruct((1, E), F32),
            jax.ShapeDtypeStruct((1, E), F32),
        ],
        scratch_shapes=[pltpu.VMEM((1, E), F32), pltpu.VMEM((1, E), F32)],
    )(scores)
    dest, blk_e, lb = pl.pallas_call(
        _k4b_body,
        in_specs=[
            pl.BlockSpec((1, E), lambda: (0, 0)),
            pl.BlockSpec((1, E), lambda: (0, 0)),
            pl.BlockSpec((S, 1), lambda: (0, 0)),
            pl.BlockSpec((S, 1), lambda: (0, 0)),
        ],
        out_specs=[
            pl.BlockSpec((S, 1), lambda: (0, 0)),
            pl.BlockSpec((NBLK, 1), lambda: (0, 0)),
            pl.BlockSpec((1, 1), lambda: (0, 0)),
        ],
        out_shape=[
            jax.ShapeDtypeStruct((S, 1), jnp.int32),
            jax.ShapeDtypeStruct((NBLK, 1), jnp.int32),
            jax.ShapeDtypeStruct((1, 1), F32),
        ],
    )(cnt, ssum, top1, rank)
    return gate, dest, blk_e, lb


def _k5_body(be_ref, xs_ref, w1_ref, b1_ref, w2_ref, b2_ref, y_out):
    h = jnp.maximum(_dot(xs_ref[...], w1_ref[0]) + b1_ref[0], 0.0)
    y_out[...] = _dot(h, w2_ref[0]) + b2_ref[0]


def _moe_ffn(blk_e, xs, w1, b1, w2, b2):
    grid_spec = pltpu.PrefetchScalarGridSpec(
        num_scalar_prefetch=1,
        grid=(NBLK,),
        in_specs=[
            pl.BlockSpec((TB, H), lambda b, be: (b, 0)),
            pl.BlockSpec((1, FF, H), lambda b, be: (be[b], 0, 0)),
            pl.BlockSpec((1, 1, FF), lambda b, be: (be[b], 0, 0)),
            pl.BlockSpec((1, H, FF), lambda b, be: (be[b], 0, 0)),
            pl.BlockSpec((1, 1, H), lambda b, be: (be[b], 0, 0)),
        ],
        out_specs=pl.BlockSpec((TB, H), lambda b, be: (b, 0)),
    )
    return pl.pallas_call(
        _k5_body,
        grid_spec=grid_spec,
        out_shape=jax.ShapeDtypeStruct((PADDED, H), F32),
    )(blk_e, xs, w1, b1.reshape(E, 1, FF), w2, b2.reshape(E, 1, H))


def _k7_body(x_ref, w_ref, o_ref):
    o_ref[...] = jax.lax.dot_general(
        x_ref[...], w_ref[...], (((1,), (1,)), ((), ())),
        preferred_element_type=F32)


def _lm_head(xbf, w_bf16):
    return pl.pallas_call(
        _k7_body,
        grid=(V // VB, S // RB),
        in_specs=[
            pl.BlockSpec((RB, H), lambda j, i: (i, 0)),
            pl.BlockSpec((VB, H), lambda j, i: (j, 0)),
        ],
        out_specs=pl.BlockSpec((RB, VB), lambda j, i: (i, j)),
        out_shape=jax.ShapeDtypeStruct((S, V), F32),
    )(xbf, w_bf16)


def kernel(input_ids, embed, pos_embed, in_proj_w, in_proj_b, out_proj_w,
           out_proj_b, ln1_g, ln1_b, gate_w, w1, b1, w2, b2, ln2_g, ln2_b,
           ln_f_g, ln_f_b, lm_head_w):
    ids = input_ids.reshape(S).astype(jnp.int32)
    emb = _sc_gather(embed, ids)
    x = emb + pos_embed
    lbs = []
    for l in range(L):
        qkv = _mm(x, in_proj_w[l], in_proj_b[l])
        q, k, v = jnp.split(qkv, 3, axis=-1)
        qh = q.reshape(S, NH, DH).transpose(1, 0, 2)
        kh = k.reshape(S, NH, DH).transpose(1, 0, 2)
        vh = v.reshape(1, S, NH, DH).transpose(0, 2, 1, 3)
        s = _attn_scores(qh, kh).reshape(1, NH, S, S)
        att = jax.nn.softmax(s, axis=-1)
        a = (att @ vh).transpose(0, 2, 1, 3).reshape(S, H)
        o = _mm_nobias(a, out_proj_w[l])
        xf = _ln(x + (o + out_proj_b[l]), ln1_g[l], ln1_b[l])
        glog = _mm_nobias(xf, gate_w[l])
        scores = jax.nn.softmax(glog, axis=-1)
        gate, dest, blk_e, lb = _route(scores)
        lbs.append(lb[0, 0])
        d = dest[:, 0]
        xs = _sc_scatter(xf, d, PADDED)
        y = _moe_ffn(blk_e[:, 0], xs, w1[l], b1[l], w2[l], b2[l])
        ym = _sc_gather(y, d)
        m = jax.lax.optimization_barrier(gate * ym)
        x = _ln(xf + m, ln2_g[l], ln2_b[l])
    x = _ln(x, ln_f_g, ln_f_b)
    logits = _lm_head(x.astype(jnp.bfloat16), lm_head_w.astype(jnp.bfloat16))
    lb_mean = jnp.stack(lbs).mean()
    return logits.reshape(1, S, V), lb_mean
